# baseline (speedup 1.0000x reference)
import jax
import jax.numpy as jnp
from jax.experimental import pallas as pl
from jax.experimental.pallas import tpu as pltpu

NUM_TOKENS = 32768
HIDDEN = 2048
NUM_EXPERTS = 64
TOP_K = 8
BLOCK = 1024
N_BLOCKS = NUM_TOKENS // BLOCK
N_COPY_CHUNKS = 8
COPY_ROWS = NUM_TOKENS // N_COPY_CHUNKS


def _fused_kernel(h_ref, logits_ref, h_out_ref, w_ref, id_ref, copy_sem):
    i = pl.program_id(0)

    @pl.when(i == 0)
    def _start_copies():
        for c in range(N_COPY_CHUNKS):
            pltpu.make_async_copy(
                h_ref.at[pl.ds(c * COPY_ROWS, COPY_ROWS), :],
                h_out_ref.at[pl.ds(c * COPY_ROWS, COPY_ROWS), :],
                copy_sem,
            ).start()

    x = logits_ref[...]
    b = x.shape[0]
    col8 = jax.lax.broadcasted_iota(jnp.int32, (b, TOP_K), 1)
    lane = jax.lax.broadcasted_iota(jnp.int32, (b, NUM_EXPERTS), 1)
    vals = jnp.zeros((b, TOP_K), dtype=jnp.float32)
    ids = jnp.zeros((b, TOP_K), dtype=jnp.int32)
    cur = x
    for j in range(TOP_K):
        m = jnp.max(cur, axis=-1, keepdims=True)
        a = jnp.argmax(cur, axis=-1).astype(jnp.int32)
        a2 = a[:, None]
        vals = jnp.where(col8 == j, m, vals)
        ids = jnp.where(col8 == j, a2, ids)
        cur = jnp.where(lane == a2, -jnp.inf, cur)
    e = jnp.exp(vals - vals[:, 0:1])
    w_ref[...] = e / jnp.sum(e, axis=-1, keepdims=True)
    id_ref[...] = ids

    @pl.when(i == N_BLOCKS - 1)
    def _drain_copies():
        for c in range(N_COPY_CHUNKS):
            pltpu.make_async_copy(
                h_ref.at[pl.ds(c * COPY_ROWS, COPY_ROWS), :],
                h_out_ref.at[pl.ds(c * COPY_ROWS, COPY_ROWS), :],
                copy_sem,
            ).wait()


def kernel(hidden_states, router_logits):
    grid = (N_BLOCKS,)
    h_out, topk_weights, topk_ids = pl.pallas_call(
        _fused_kernel,
        grid=grid,
        in_specs=[
            pl.BlockSpec(memory_space=pl.ANY),
            pl.BlockSpec((BLOCK, NUM_EXPERTS), lambda i: (i, 0)),
        ],
        out_specs=[
            pl.BlockSpec(memory_space=pl.ANY),
            pl.BlockSpec((BLOCK, TOP_K), lambda i: (i, 0)),
            pl.BlockSpec((BLOCK, TOP_K), lambda i: (i, 0)),
        ],
        out_shape=[
            jax.ShapeDtypeStruct((NUM_TOKENS, HIDDEN), jnp.float32),
            jax.ShapeDtypeStruct((NUM_TOKENS, TOP_K), jnp.float32),
            jax.ShapeDtypeStruct((NUM_TOKENS, TOP_K), jnp.int32),
        ],
        scratch_shapes=[pltpu.SemaphoreType.DMA],
    )(hidden_states, router_logits)
    return h_out, topk_weights, topk_ids


# speedup vs baseline: 37.9578x; 37.9578x over previous
import jax
import jax.numpy as jnp
from jax.experimental import pallas as pl
from jax.experimental.pallas import tpu as pltpu

NUM_TOKENS = 32768
HIDDEN = 2048
NUM_EXPERTS = 64
TOP_K = 8
BLOCK = 1024
N_BLOCKS = NUM_TOKENS // BLOCK


def _fused_kernel(h_ref, logits_ref, h_out_ref, w_ref, id_ref, copy_sem):
    fwd = pltpu.make_async_copy(h_ref, h_out_ref, copy_sem)
    fwd.start()

    x = logits_ref[...]
    xt = x.T
    b = xt.shape[1]
    mx = jnp.max(xt, axis=0, keepdims=True)
    e = jnp.exp(xt - mx)
    probs = e / jnp.sum(e, axis=0, keepdims=True)

    row8 = jax.lax.broadcasted_iota(jnp.int32, (TOP_K, b), 0)
    row64 = jax.lax.broadcasted_iota(jnp.int32, (NUM_EXPERTS, b), 0)
    vals = jnp.zeros((TOP_K, b), dtype=jnp.float32)
    ids = jnp.zeros((TOP_K, b), dtype=jnp.int32)
    cur = probs
    for j in range(TOP_K):
        m = jnp.max(cur, axis=0, keepdims=True)
        a = jnp.argmax(cur, axis=0).astype(jnp.int32)
        a2 = a[None, :]
        vals = jnp.where(row8 == j, m, vals)
        ids = jnp.where(row8 == j, a2, ids)
        cur = jnp.where(row64 == a2, -1.0, cur)
    w = vals / jnp.sum(vals, axis=0, keepdims=True)
    w_ref[...] = w.T
    id_ref[...] = ids.T

    fwd.wait()


def kernel(hidden_states, router_logits):
    grid = (N_BLOCKS,)
    h_out, topk_weights, topk_ids = pl.pallas_call(
        _fused_kernel,
        grid=grid,
        in_specs=[
            pl.BlockSpec((BLOCK, HIDDEN), lambda i: (i, 0)),
            pl.BlockSpec((BLOCK, NUM_EXPERTS), lambda i: (i, 0)),
        ],
        out_specs=[
            pl.BlockSpec((BLOCK, HIDDEN), lambda i: (i, 0)),
            pl.BlockSpec((BLOCK, TOP_K), lambda i: (i, 0)),
            pl.BlockSpec((BLOCK, TOP_K), lambda i: (i, 0)),
        ],
        out_shape=[
            jax.ShapeDtypeStruct((NUM_TOKENS, HIDDEN), jnp.float32),
            jax.ShapeDtypeStruct((NUM_TOKENS, TOP_K), jnp.float32),
            jax.ShapeDtypeStruct((NUM_TOKENS, TOP_K), jnp.int32),
        ],
        scratch_shapes=[pltpu.SemaphoreType.DMA],
    )(hidden_states, router_logits)
    return h_out, topk_weights, topk_ids


# speedup vs baseline: 37.9715x; 1.0004x over previous
import jax
import jax.numpy as jnp
from jax.experimental import pallas as pl
from jax.experimental.pallas import tpu as pltpu

NUM_TOKENS = 32768
HIDDEN = 2048
NUM_EXPERTS = 64
TOP_K = 8
BLOCK = 1024
N_BLOCKS = NUM_TOKENS // BLOCK


def _fused_kernel(h_ref, logits_ref, h_out_ref, w_ref, id_ref, copy_sem):
    h_out_ref[...] = h_ref[...]

    x = logits_ref[...]
    xt = x.T
    b = xt.shape[1]
    mx = jnp.max(xt, axis=0, keepdims=True)
    e = jnp.exp(xt - mx)
    probs = e / jnp.sum(e, axis=0, keepdims=True)

    row8 = jax.lax.broadcasted_iota(jnp.int32, (TOP_K, b), 0)
    row64 = jax.lax.broadcasted_iota(jnp.int32, (NUM_EXPERTS, b), 0)
    vals = jnp.zeros((TOP_K, b), dtype=jnp.float32)
    ids = jnp.zeros((TOP_K, b), dtype=jnp.int32)
    cur = probs
    for j in range(TOP_K):
        m = jnp.max(cur, axis=0, keepdims=True)
        a = jnp.argmax(cur, axis=0).astype(jnp.int32)
        a2 = a[None, :]
        vals = jnp.where(row8 == j, m, vals)
        ids = jnp.where(row8 == j, a2, ids)
        cur = jnp.where(row64 == a2, -1.0, cur)
    w = vals / jnp.sum(vals, axis=0, keepdims=True)
    w_ref[...] = w.T
    id_ref[...] = ids.T


def kernel(hidden_states, router_logits):
    grid = (N_BLOCKS,)
    h_out, topk_weights, topk_ids = pl.pallas_call(
        _fused_kernel,
        grid=grid,
        in_specs=[
            pl.BlockSpec((BLOCK, HIDDEN), lambda i: (i, 0)),
            pl.BlockSpec((BLOCK, NUM_EXPERTS), lambda i: (i, 0)),
        ],
        out_specs=[
            pl.BlockSpec((BLOCK, HIDDEN), lambda i: (i, 0)),
            pl.BlockSpec((BLOCK, TOP_K), lambda i: (i, 0)),
            pl.BlockSpec((BLOCK, TOP_K), lambda i: (i, 0)),
        ],
        out_shape=[
            jax.ShapeDtypeStruct((NUM_TOKENS, HIDDEN), jnp.float32),
            jax.ShapeDtypeStruct((NUM_TOKENS, TOP_K), jnp.float32),
            jax.ShapeDtypeStruct((NUM_TOKENS, TOP_K), jnp.int32),
        ],
        scratch_shapes=[pltpu.SemaphoreType.DMA],
    )(hidden_states, router_logits)
    return h_out, topk_weights, topk_ids


# speedup vs baseline: 38.7858x; 1.0214x over previous
import jax
import jax.numpy as jnp
from jax.experimental import pallas as pl
from jax.experimental.pallas import tpu as pltpu

NUM_TOKENS = 32768
HIDDEN = 2048
NUM_EXPERTS = 64
TOP_K = 8
BLOCK = 1536
N_BLOCKS = NUM_TOKENS // BLOCK


def _fused_kernel(h_ref, logits_ref, h_out_ref, w_ref, id_ref, copy_sem):
    h_out_ref[...] = h_ref[...]

    x = logits_ref[...]
    xt = x.T
    b = xt.shape[1]
    mx = jnp.max(xt, axis=0, keepdims=True)
    e = jnp.exp(xt - mx)
    probs = e / jnp.sum(e, axis=0, keepdims=True)

    row8 = jax.lax.broadcasted_iota(jnp.int32, (TOP_K, b), 0)
    row64 = jax.lax.broadcasted_iota(jnp.int32, (NUM_EXPERTS, b), 0)
    vals = jnp.zeros((TOP_K, b), dtype=jnp.float32)
    ids = jnp.zeros((TOP_K, b), dtype=jnp.int32)
    cur = probs
    for j in range(TOP_K):
        m = jnp.max(cur, axis=0, keepdims=True)
        a = jnp.argmax(cur, axis=0).astype(jnp.int32)
        a2 = a[None, :]
        vals = jnp.where(row8 == j, m, vals)
        ids = jnp.where(row8 == j, a2, ids)
        cur = jnp.where(row64 == a2, -1.0, cur)
    w = vals / jnp.sum(vals, axis=0, keepdims=True)
    w_ref[...] = w.T
    id_ref[...] = ids.T


def kernel(hidden_states, router_logits):
    grid = (N_BLOCKS,)
    h_out, topk_weights, topk_ids = pl.pallas_call(
        _fused_kernel,
        grid=grid,
        in_specs=[
            pl.BlockSpec((BLOCK, HIDDEN), lambda i: (i, 0)),
            pl.BlockSpec((BLOCK, NUM_EXPERTS), lambda i: (i, 0)),
        ],
        out_specs=[
            pl.BlockSpec((BLOCK, HIDDEN), lambda i: (i, 0)),
            pl.BlockSpec((BLOCK, TOP_K), lambda i: (i, 0)),
            pl.BlockSpec((BLOCK, TOP_K), lambda i: (i, 0)),
        ],
        out_shape=[
            jax.ShapeDtypeStruct((NUM_TOKENS, HIDDEN), jnp.float32),
            jax.ShapeDtypeStruct((NUM_TOKENS, TOP_K), jnp.float32),
            jax.ShapeDtypeStruct((NUM_TOKENS, TOP_K), jnp.int32),
        ],
        scratch_shapes=[pltpu.SemaphoreType.DMA],
    )(hidden_states, router_logits)
    return h_out, topk_weights, topk_ids
